# baseline (speedup 1.0000x reference)
import functools

import jax
import jax.numpy as jnp
from jax import lax
from jax.experimental import pallas as pl
from jax.experimental.pallas import tpu as pltpu
from jax.experimental.pallas import tpu_sc as plsc

_D = 64
_S = 50
_NB = 16
_NC = 2
_NS = 16
_NW = _NC * _NS


@jax.jit
def _sc_gather(table, idx):
    b, s = idx.shape
    b_per_w = b // _NW
    nchunks = b_per_w // _NB
    mesh = plsc.VectorSubcoreMesh(core_axis_name="c", subcore_axis_name="s")

    @functools.partial(
        pl.kernel,
        out_type=jax.ShapeDtypeStruct((b, s, _D), jnp.float32),
        mesh=mesh,
        scratch_types=[
            pltpu.VMEM((_NB, _S), jnp.int32),
            pltpu.VMEM((_NB, _S, _D), jnp.float32),
            pltpu.SemaphoreType.DMA,
        ],
        compiler_params=pltpu.CompilerParams(use_tc_tiling_on_sc=False),
    )
    def k(table_hbm, idx_hbm, out_hbm, idx_v, rows_v, gsem):
        wid = lax.axis_index("s") * _NC + lax.axis_index("c")
        base = wid * b_per_w

        def body(c, carry):
            b0 = base + c * _NB
            pltpu.sync_copy(idx_hbm.at[pl.ds(b0, _NB)], idx_v)
            copies = [
                pltpu.async_copy(
                    table_hbm.at[idx_v.at[j]],
                    rows_v.at[j],
                    gsem,
                )
                for j in range(_NB)
            ]
            for cp in copies:
                cp.wait()
            pltpu.sync_copy(rows_v, out_hbm.at[pl.ds(b0, _NB)])
            return carry

        lax.fori_loop(0, nchunks, body, 0)

    return k(table, idx)


def kernel(input_tensor, indices):
    return _sc_gather(input_tensor, indices.astype(jnp.int32))
